# baseline (speedup 1.0000x reference)
import functools

import jax
import jax.numpy as jnp
from jax import lax
from jax.experimental import pallas as pl
from jax.experimental.pallas import tpu as pltpu
from jax.experimental.pallas import tpu_sc as plsc

VOCAB = 1000000
EMB_DIM = 4
BATCH = 16384
NUM_SLOTS = 3

L = 16
NW = 32
B_PER_W = BATCH // NW
CHUNK = 128
CHUNKS_PER_W = B_PER_W // CHUNK
GATHERS = NUM_SLOTS * CHUNKS_PER_W
GROUPS = B_PER_W // L
GPC = CHUNK // L


def _sc_forward(xt_hbm, embf_hbm, wb_hbm, out_hbm, idx_v, *rest):
    eidx = rest[0:GATHERS]
    vals = rest[GATHERS : 2 * GATHERS]
    wv, out_v, sem = rest[2 * GATHERS :]
    nc = 2
    wid = lax.axis_index("s") * nc + lax.axis_index("c")
    base_chunk = wid * CHUNKS_PER_W

    for s in range(NUM_SLOTS):
        pltpu.sync_copy(
            xt_hbm.at[s, pl.ds(base_chunk, CHUNKS_PER_W)],
            idx_v.at[pl.ds(s * CHUNKS_PER_W, CHUNKS_PER_W)],
        )
    pltpu.sync_copy(wb_hbm, wv)

    descs = []
    for t in range(GATHERS):
        for g in range(GPC):
            xi = idx_v[t, pl.ds(g * L, L)]
            x4 = xi * EMB_DIM
            for c in range(EMB_DIM):
                eidx[t][pl.ds(c * CHUNK + g * L, L)] = x4 + c
        descs.append(pltpu.async_copy(embf_hbm.at[eidx[t]], vals[t], sem))
    for d in descs:
        d.wait()

    gw = [wv[i] for i in range(9)]
    dw = [wv[9 + c] for c in range(EMB_DIM)]
    bias = wv[13]
    one = jnp.ones((L,), jnp.float32)

    for g in range(GROUPS):
        chunk = g // GPC
        off = (g % GPC) * L
        xf = []
        for s in range(NUM_SLOTS):
            xi = idx_v[s * CHUNKS_PER_W + chunk, pl.ds(off, L)]
            xf.append(xi.astype(jnp.float32))
        logits = []
        for j in range(NUM_SLOTS):
            l = xf[0] * gw[j]
            for s in range(1, NUM_SLOTS):
                l = l + xf[s] * gw[s * NUM_SLOTS + j]
            logits.append(l)
        m = jnp.maximum(logits[0], jnp.maximum(logits[1], logits[2]))
        e = [jnp.exp(l - m) for l in logits]
        denom = e[0] + e[1] + e[2]
        acc = None
        for s in range(NUM_SLOTS):
            v = vals[s * CHUNKS_PER_W + chunk]
            dot = None
            for c in range(EMB_DIM):
                term = v[pl.ds(c * CHUNK + off, L)] * dw[c]
                dot = term if dot is None else dot + term
            gdot = e[s] * dot
            acc = gdot if acc is None else acc + gdot
        z = acc / denom + bias
        out_v[pl.ds(g * L, L)] = one / (one + jnp.exp(-z))

    pltpu.sync_copy(out_v, out_hbm.at[pl.ds(wid * B_PER_W, B_PER_W)])


def kernel(x, emb_table, gate_W, dense_W, dense_b):
    xt = x.astype(jnp.int32).T.reshape(NUM_SLOTS, BATCH // CHUNK, CHUNK)
    unit = lax.optimization_barrier(jnp.ones((), jnp.float32))
    embf = emb_table.reshape(-1) * unit
    scalars = jnp.concatenate(
        [
            gate_W.astype(jnp.float32).reshape(-1),
            dense_W.astype(jnp.float32).reshape(-1),
            dense_b.astype(jnp.float32).reshape(-1),
            jnp.zeros((2,), jnp.float32),
        ]
    )
    wb = jnp.broadcast_to(scalars[:, None], (16, L))

    mesh = plsc.VectorSubcoreMesh(core_axis_name="c", subcore_axis_name="s")
    fwd = functools.partial(
        pl.kernel,
        mesh=mesh,
        compiler_params=pltpu.CompilerParams(
            needs_layout_passes=False, use_tc_tiling_on_sc=False
        ),
        out_type=jax.ShapeDtypeStruct((BATCH,), jnp.float32),
        scratch_types=(
            [pltpu.VMEM((GATHERS, CHUNK), jnp.int32)]
            + [pltpu.VMEM((CHUNK * EMB_DIM,), jnp.int32) for _ in range(GATHERS)]
            + [pltpu.VMEM((CHUNK * EMB_DIM,), jnp.float32) for _ in range(GATHERS)]
            + [
                pltpu.VMEM((16, L), jnp.float32),
                pltpu.VMEM((B_PER_W,), jnp.float32),
                pltpu.SemaphoreType.DMA,
            ]
        ),
    )(_sc_forward)
    out = fwd(xt, embf, wb)
    return out.reshape(BATCH, 1)


# speedup vs baseline: 19.4675x; 19.4675x over previous
import functools

import jax
import jax.numpy as jnp
from jax import lax
from jax.experimental import pallas as pl
from jax.experimental.pallas import tpu as pltpu
from jax.experimental.pallas import tpu_sc as plsc

VOCAB = 1000000
EMB_DIM = 4
BATCH = 16384
NUM_SLOTS = 3

L = 16
NW = 32
B_PER_W = BATCH // NW
CHUNK = 128
CHUNKS_PER_W = B_PER_W // CHUNK
GATHERS = NUM_SLOTS * CHUNKS_PER_W
GROUPS = B_PER_W // L
GPC = CHUNK // L


def _sc_forward(xt_hbm, embf_hbm, wb_hbm, out_hbm, idx_v, *rest):
    eidx = rest[0:GATHERS]
    vals = rest[GATHERS : 2 * GATHERS]
    wv, out_v, sem = rest[2 * GATHERS :]
    nc = 2
    wid = lax.axis_index("s") * nc + lax.axis_index("c")
    base_chunk = wid * CHUNKS_PER_W

    for s in range(NUM_SLOTS):
        pltpu.sync_copy(
            xt_hbm.at[s, pl.ds(base_chunk, CHUNKS_PER_W)],
            idx_v.at[pl.ds(s * CHUNKS_PER_W, CHUNKS_PER_W)],
        )
    pltpu.sync_copy(wb_hbm, wv)

    descs = []
    for t in range(GATHERS):
        for g in range(GPC):
            xi = idx_v[t, pl.ds(g * L, L)]
            for c in range(EMB_DIM):
                eidx[t][pl.ds(c * CHUNK + g * L, L)] = xi + c * VOCAB
        descs.append(pltpu.async_copy(embf_hbm.at[eidx[t]], vals[t], sem))
    for d in descs:
        d.wait()

    gw = [wv[i] for i in range(9)]
    dw = [wv[9 + c] for c in range(EMB_DIM)]
    bias = wv[13]
    one = jnp.ones((L,), jnp.float32)

    for g in range(GROUPS):
        chunk = g // GPC
        off = (g % GPC) * L
        xf = []
        for s in range(NUM_SLOTS):
            xi = idx_v[s * CHUNKS_PER_W + chunk, pl.ds(off, L)]
            xf.append(xi.astype(jnp.float32))
        logits = []
        for j in range(NUM_SLOTS):
            l = xf[0] * gw[j]
            for s in range(1, NUM_SLOTS):
                l = l + xf[s] * gw[s * NUM_SLOTS + j]
            logits.append(l)
        m = jnp.maximum(logits[0], jnp.maximum(logits[1], logits[2]))
        e = [jnp.exp(l - m) for l in logits]
        denom = e[0] + e[1] + e[2]
        acc = None
        for s in range(NUM_SLOTS):
            v = vals[s * CHUNKS_PER_W + chunk]
            dot = None
            for c in range(EMB_DIM):
                term = v[pl.ds(c * CHUNK + off, L)] * dw[c]
                dot = term if dot is None else dot + term
            gdot = e[s] * dot
            acc = gdot if acc is None else acc + gdot
        z = acc / denom + bias
        out_v[pl.ds(g * L, L)] = one / (one + jnp.exp(-z))

    pltpu.sync_copy(out_v, out_hbm.at[pl.ds(wid * B_PER_W, B_PER_W)])


def kernel(x, emb_table, gate_W, dense_W, dense_b):
    xt = x.astype(jnp.int32).T.reshape(NUM_SLOTS, BATCH // CHUNK, CHUNK)
    embf = emb_table.T.reshape(-1)
    scalars = jnp.concatenate(
        [
            gate_W.astype(jnp.float32).reshape(-1),
            dense_W.astype(jnp.float32).reshape(-1),
            dense_b.astype(jnp.float32).reshape(-1),
            jnp.zeros((2,), jnp.float32),
        ]
    )
    wb = jnp.broadcast_to(scalars[:, None], (16, L))

    mesh = plsc.VectorSubcoreMesh(core_axis_name="c", subcore_axis_name="s")
    fwd = functools.partial(
        pl.kernel,
        mesh=mesh,
        compiler_params=pltpu.CompilerParams(
            needs_layout_passes=False, use_tc_tiling_on_sc=False
        ),
        out_type=jax.ShapeDtypeStruct((BATCH,), jnp.float32),
        scratch_types=(
            [pltpu.VMEM((GATHERS, CHUNK), jnp.int32)]
            + [pltpu.VMEM((CHUNK * EMB_DIM,), jnp.int32) for _ in range(GATHERS)]
            + [pltpu.VMEM((CHUNK * EMB_DIM,), jnp.float32) for _ in range(GATHERS)]
            + [
                pltpu.VMEM((16, L), jnp.float32),
                pltpu.VMEM((B_PER_W,), jnp.float32),
                pltpu.SemaphoreType.DMA,
            ]
        ),
    )(_sc_forward)
    out = fwd(xt, embf, wb)
    return out.reshape(BATCH, 1)


# speedup vs baseline: 29.5676x; 1.5188x over previous
import functools

import jax
import jax.numpy as jnp
from jax import lax
from jax.experimental import pallas as pl
from jax.experimental.pallas import tpu as pltpu
from jax.experimental.pallas import tpu_sc as plsc
from jax.experimental import layout as jexp_layout

VOCAB = 1000000
EMB_DIM = 4
BATCH = 16384
NUM_SLOTS = 3

L = 16
NW = 32
B_PER_W = BATCH // NW
CHUNK = 128
CHUNKS_PER_W = B_PER_W // CHUNK
GATHERS = NUM_SLOTS * CHUNKS_PER_W
GROUPS = B_PER_W // L
GPC = CHUNK // L


MAIN_ROWS = 999936
TAIL_ROWS = VOCAB - MAIN_ROWS


def _sc_forward(xt_hbm, embf_hbm, tail_hbm, wb_hbm, out_hbm, idx_v, *rest):
    eidx = rest[0:GATHERS]
    vals = rest[GATHERS : 2 * GATHERS]
    wv, tail_v, tdot_v, out_v, sem = rest[2 * GATHERS :]
    nc = 2
    wid = lax.axis_index("s") * nc + lax.axis_index("c")
    base_chunk = wid * CHUNKS_PER_W

    for s in range(NUM_SLOTS):
        pltpu.sync_copy(
            xt_hbm.at[s, pl.ds(base_chunk, CHUNKS_PER_W)],
            idx_v.at[pl.ds(s * CHUNKS_PER_W, CHUNKS_PER_W)],
        )
    pltpu.sync_copy(wb_hbm, wv)
    pltpu.sync_copy(tail_hbm, tail_v)

    descs = []
    seven = jnp.full((L,), 127, jnp.int32)
    maxi = jnp.full((L,), MAIN_ROWS - 1, jnp.int32)
    for t in range(GATHERS):
        for g in range(GPC):
            xi = idx_v[t, pl.ds(g * L, L)]
            xc = jnp.minimum(xi, maxi)
            hi = lax.shift_left(lax.shift_right_logical(xc, 7), 9) + (xc & seven)
            for c in range(EMB_DIM):
                eidx[t][pl.ds(c * CHUNK + g * L, L)] = hi + c * CHUNK
        descs.append(pltpu.async_copy(embf_hbm.at[eidx[t]], vals[t], sem))
    for d in descs:
        d.wait()

    gw = [wv[i] for i in range(9)]
    dw = [wv[9 + c] for c in range(EMB_DIM)]
    bias = wv[13]
    one = jnp.ones((L,), jnp.float32)
    tmin = jnp.full((L,), MAIN_ROWS, jnp.int32)
    zero = jnp.zeros((L,), jnp.int32)

    for g in range(TAIL_ROWS // L):
        td = None
        for c in range(EMB_DIM):
            term = tail_v[pl.ds(c * TAIL_ROWS + g * L, L)] * dw[c]
            td = term if td is None else td + term
        tdot_v[pl.ds(g * L, L)] = td

    for g in range(GROUPS):
        chunk = g // GPC
        off = (g % GPC) * L
        xi_s, xf = [], []
        for s in range(NUM_SLOTS):
            xi = idx_v[s * CHUNKS_PER_W + chunk, pl.ds(off, L)]
            xi_s.append(xi)
            xf.append(xi.astype(jnp.float32))
        logits = []
        for j in range(NUM_SLOTS):
            l = xf[0] * gw[j]
            for s in range(1, NUM_SLOTS):
                l = l + xf[s] * gw[s * NUM_SLOTS + j]
            logits.append(l)
        m = jnp.maximum(logits[0], jnp.maximum(logits[1], logits[2]))
        e = [jnp.exp(l - m) for l in logits]
        denom = e[0] + e[1] + e[2]
        acc = None
        for s in range(NUM_SLOTS):
            v = vals[s * CHUNKS_PER_W + chunk]
            dot = None
            for c in range(EMB_DIM):
                term = v[pl.ds(c * CHUNK + off, L)] * dw[c]
                dot = term if dot is None else dot + term
            ti = jnp.maximum(xi_s[s] - tmin, zero)
            tdot = plsc.load_gather(tdot_v, [ti])
            dot = jnp.where(xi_s[s] >= tmin, tdot, dot)
            gdot = e[s] * dot
            acc = gdot if acc is None else acc + gdot
        z = acc / denom + bias
        out_v[pl.ds(g * L, L)] = one / (one + jnp.exp(-z))

    pltpu.sync_copy(out_v, out_hbm.at[pl.ds(wid * B_PER_W, B_PER_W)])


def kernel(x, emb_table, gate_W, dense_W, dense_b):
    xt = x.astype(jnp.int32).T.reshape(NUM_SLOTS, BATCH // CHUNK, CHUNK)
    embv = (
        emb_table[:MAIN_ROWS]
        .T.reshape(EMB_DIM, MAIN_ROWS // CHUNK, CHUNK)
        .transpose(1, 0, 2)
    )
    embv = jexp_layout.with_layout_constraint(
        embv,
        jexp_layout.Layout(major_to_minor=(0, 1, 2), tiling=((4, 128),)),
    )
    embf = embv.reshape(-1)
    tail = emb_table[MAIN_ROWS:].T.reshape(-1)
    scalars = jnp.concatenate(
        [
            gate_W.astype(jnp.float32).reshape(-1),
            dense_W.astype(jnp.float32).reshape(-1),
            dense_b.astype(jnp.float32).reshape(-1),
            jnp.zeros((2,), jnp.float32),
        ]
    )
    wb = jnp.broadcast_to(scalars[:, None], (16, L))

    mesh = plsc.VectorSubcoreMesh(core_axis_name="c", subcore_axis_name="s")
    fwd = functools.partial(
        pl.kernel,
        mesh=mesh,
        compiler_params=pltpu.CompilerParams(
            needs_layout_passes=False, use_tc_tiling_on_sc=False
        ),
        out_type=jax.ShapeDtypeStruct((BATCH,), jnp.float32),
        scratch_types=(
            [pltpu.VMEM((GATHERS, CHUNK), jnp.int32)]
            + [pltpu.VMEM((CHUNK * EMB_DIM,), jnp.int32) for _ in range(GATHERS)]
            + [pltpu.VMEM((CHUNK * EMB_DIM,), jnp.float32) for _ in range(GATHERS)]
            + [
                pltpu.VMEM((16, L), jnp.float32),
                pltpu.VMEM((TAIL_ROWS * EMB_DIM,), jnp.float32),
                pltpu.VMEM((TAIL_ROWS,), jnp.float32),
                pltpu.VMEM((B_PER_W,), jnp.float32),
                pltpu.SemaphoreType.DMA,
            ]
        ),
    )(_sc_forward)
    out = fwd(xt, embf, tail, wb)
    return out.reshape(BATCH, 1)
